# baseline (speedup 1.0000x reference)
import functools

import jax
import jax.numpy as jnp
from jax import lax
from jax.experimental import pallas as pl
from jax.experimental.pallas import tpu as pltpu
from jax.experimental.pallas import tpu_sc as plsc

N = 10000
NPAD = 10240
E = 320000
NC = 2
NS = 16
NW = NC * NS
K = 128
CH = 80
EPW = CH * K
PE = NW * EPW
SLICE = NPAD // NS
D_HID = 64
N_CLASSES = 16


KD = 128
CHD = EPW // KD


def _deg_body(dstb, wb, out, acc, dstv, wv, zv, sem):
    c = lax.axis_index("c")
    s = lax.axis_index("s")
    wid = c * NS + s

    def z(j, _):
        zv[pl.ds(j * 16, 16)] = jnp.zeros((16,), jnp.float32)
        return 0

    lax.fori_loop(0, SLICE // 16, z, 0)
    pltpu.sync_copy(zv, acc.at[pl.ds(s * SLICE, SLICE)])
    plsc.subcore_barrier()

    pltpu.sync_copy(dstb.at[wid], dstv)
    pltpu.sync_copy(wb.at[wid], wv)

    def fire(i, _):
        pltpu.async_copy(wv.at[i], acc.at[dstv.at[i]], sem, add=True)
        return 0

    lax.fori_loop(0, CHD, fire, 0)

    def drain(i, _):
        pltpu.make_async_copy(wv.at[i], acc.at[dstv.at[i]], sem).wait()
        return 0

    lax.fori_loop(0, CHD, drain, 0)
    plsc.subcore_barrier()
    pltpu.sync_copy(acc.at[pl.ds(s * SLICE, SLICE)],
                    out.at[c, pl.ds(s * SLICE, SLICE)])


_deg = pl.kernel(
    _deg_body,
    out_type=jax.ShapeDtypeStruct((NC, NPAD), jnp.float32),
    mesh=plsc.VectorSubcoreMesh(core_axis_name="c", subcore_axis_name="s"),
    compiler_params=pltpu.CompilerParams(use_tc_tiling_on_sc=False),
    scratch_types=[
        pltpu.VMEM_SHARED((NPAD,), jnp.float32),
        pltpu.VMEM((CHD, KD), jnp.int32),
        pltpu.VMEM((CHD, KD), jnp.float32),
        pltpu.VMEM((SLICE,), jnp.float32),
        pltpu.SemaphoreType.DMA,
    ],
)


def _agg_body(D, g_hbm, srcb, dstb, wb, out, acc, srcv, dstv, wv,
              rows0, rows1, gsem, ssem):
    c = lax.axis_index("c")
    s = lax.axis_index("s")
    wid = c * NS + s

    def zr(j, _):
        for t in range(D // 16):
            rows0[j, pl.ds(t * 16, 16)] = jnp.zeros((16,), jnp.float32)
        return 0

    lax.fori_loop(0, K, zr, 0)

    def zc(r, _):
        pltpu.sync_copy(rows0, acc.at[pl.ds(s * SLICE + r * K, K), :])
        return 0

    lax.fori_loop(0, SLICE // K, zc, 0)
    plsc.subcore_barrier()

    pltpu.sync_copy(srcb.at[wid], srcv)
    pltpu.sync_copy(dstb.at[wid], dstv)
    pltpu.sync_copy(wb.at[wid], wv)

    def start_gather(i, buf):
        pltpu.async_copy(g_hbm.at[srcv.at[i]], buf, gsem)

    def wait_gather(i, buf):
        pltpu.make_async_copy(g_hbm.at[srcv.at[i]], buf, gsem).wait()

    def start_scatter(i, buf):
        pltpu.async_copy(buf, acc.at[dstv.at[i]], ssem, add=True)

    def wait_scatter(i, buf):
        pltpu.make_async_copy(buf, acc.at[dstv.at[i]], ssem).wait()

    def scale(i, buf):
        def grp(gidx, _):
            wvec = wv[i, pl.ds(gidx * 16, 16)]
            for l in range(16):
                nv = wvec[l]
                j = gidx * 16 + l
                for t in range(D // 16):
                    sl = pl.ds(t * 16, 16)
                    buf[j, sl] = buf[j, sl] * nv
            return 0

        lax.fori_loop(0, K // 16, grp, 0)

    start_gather(0, rows0)

    def pair(p, _):
        i0 = 2 * p
        i1 = i0 + 1
        start_gather(i1, rows1)
        wait_gather(i0, rows0)
        scale(i0, rows0)
        start_scatter(i0, rows0)
        wait_gather(i1, rows1)
        scale(i1, rows1)
        start_scatter(i1, rows1)
        wait_scatter(i0, rows0)

        @pl.when(i0 + 2 < CH)
        def _():
            start_gather(jnp.minimum(i0 + 2, CH - 1), rows0)

        wait_scatter(i1, rows1)
        return 0

    lax.fori_loop(0, CH // 2, pair, 0)
    plsc.subcore_barrier()
    pltpu.sync_copy(acc.at[pl.ds(s * SLICE, SLICE), :],
                    out.at[c, pl.ds(s * SLICE, SLICE), :])


def _make_agg(D):
    return pl.kernel(
        functools.partial(_agg_body, D),
        out_type=jax.ShapeDtypeStruct((NC, NPAD, D), jnp.float32),
        mesh=plsc.VectorSubcoreMesh(core_axis_name="c", subcore_axis_name="s"),
        compiler_params=pltpu.CompilerParams(use_tc_tiling_on_sc=False),
        scratch_types=[
            pltpu.VMEM_SHARED((NPAD, D), jnp.float32),
            pltpu.VMEM((CH, K), jnp.int32),
            pltpu.VMEM((CH, K), jnp.int32),
            pltpu.VMEM((CH, K), jnp.float32),
            pltpu.VMEM((K, D), jnp.float32),
            pltpu.VMEM((K, D), jnp.float32),
            pltpu.SemaphoreType.DMA,
            pltpu.SemaphoreType.DMA,
        ],
    )


_agg64 = _make_agg(D_HID)
_agg16 = _make_agg(N_CLASSES)


def _mms_kernel(x_ref, w_ref, dp_ref, o_ref, dc_ref):
    dc = lax.rsqrt(dp_ref[:, 0:1] + dp_ref[:, 1:2] + 1.0)
    dc_ref[...] = dc
    o_ref[...] = (jnp.dot(x_ref[...], w_ref[...],
                          preferred_element_type=jnp.float32) * dc)


def _mm_scale(x, w, dpT, bm=1000):
    m, k = x.shape
    n = w.shape[1]
    return pl.pallas_call(
        _mms_kernel,
        grid=(m // bm,),
        in_specs=[
            pl.BlockSpec((bm, k), lambda i: (i, 0)),
            pl.BlockSpec((k, n), lambda i: (0, 0)),
            pl.BlockSpec((bm, 2), lambda i: (i, 0)),
        ],
        out_specs=[
            pl.BlockSpec((bm, n), lambda i: (i, 0)),
            pl.BlockSpec((bm, 1), lambda i: (i, 0)),
        ],
        out_shape=[
            jax.ShapeDtypeStruct((m, n), jnp.float32),
            jax.ShapeDtypeStruct((m, 1), jnp.float32),
        ],
    )(x, w, dpT)


def _comb2_kernel(p0, p1, g, d, b, w2, o):
    h = jnp.maximum((p0[...] + p1[...] + g[...]) * d[...] + b[...], 0.0)
    o[...] = jnp.dot(h, w2[...], preferred_element_type=jnp.float32) * d[...]


def _comb2(p0, p1, g, dcol, b, w2, bm=1000):
    m, k = g.shape
    n = w2.shape[1]
    return pl.pallas_call(
        _comb2_kernel,
        grid=(m // bm,),
        in_specs=[
            pl.BlockSpec((bm, k), lambda i: (i, 0)),
            pl.BlockSpec((bm, k), lambda i: (i, 0)),
            pl.BlockSpec((bm, k), lambda i: (i, 0)),
            pl.BlockSpec((bm, 1), lambda i: (i, 0)),
            pl.BlockSpec((1, k), lambda i: (0, 0)),
            pl.BlockSpec((k, n), lambda i: (0, 0)),
        ],
        out_specs=pl.BlockSpec((bm, n), lambda i: (i, 0)),
        out_shape=jax.ShapeDtypeStruct((m, n), jnp.float32),
    )(p0, p1, g, dcol, b, w2)


def _final_kernel(q0, q1, g, d, b, o):
    t = (q0[...] + q1[...] + g[...]) * d[...] + b[...]
    m = jnp.max(t, axis=1, keepdims=True)
    e = t - m
    lse = jnp.log(jnp.sum(jnp.exp(e), axis=1, keepdims=True))
    o[...] = e - lse


def _final(q0, q1, g, dcol, b, bm=1000):
    m, n = g.shape
    return pl.pallas_call(
        _final_kernel,
        grid=(m // bm,),
        in_specs=[
            pl.BlockSpec((bm, n), lambda i: (i, 0)),
            pl.BlockSpec((bm, n), lambda i: (i, 0)),
            pl.BlockSpec((bm, n), lambda i: (i, 0)),
            pl.BlockSpec((bm, 1), lambda i: (i, 0)),
            pl.BlockSpec((1, n), lambda i: (0, 0)),
        ],
        out_specs=pl.BlockSpec((bm, n), lambda i: (i, 0)),
        out_shape=jax.ShapeDtypeStruct((m, n), jnp.float32),
    )(q0, q1, g, dcol, b)


def kernel(x, edge_index, edge_attr, W1, b1, W2, b2):
    pad = PE - E
    src = jnp.concatenate(
        [edge_index[0].astype(jnp.int32), jnp.zeros((pad,), jnp.int32)]
    ).reshape(NW, CH, K)
    dst = jnp.concatenate(
        [edge_index[1].astype(jnp.int32), jnp.zeros((pad,), jnp.int32)]
    ).reshape(NW, CH, K)
    wb = jnp.concatenate(
        [edge_attr, jnp.zeros((pad,), jnp.float32)]
    ).reshape(NW, CH, K)

    degp = _deg(dst.reshape(NW, CHD, KD),
                wb.reshape(NW, CHD, KD))
    g1, dcol = _mm_scale(x, W1, degp.T[:N])
    p = _agg64(g1, src, dst, wb)
    g2 = _comb2(p[0, :N], p[1, :N], g1, dcol, b1.reshape(1, -1), W2)
    q = _agg16(g2, src, dst, wb)
    return _final(q[0, :N], q[1, :N], g2, dcol, b2.reshape(1, -1))


# speedup vs baseline: 1.1889x; 1.1889x over previous
import functools

import jax
import jax.numpy as jnp
from jax import lax
from jax.experimental import pallas as pl
from jax.experimental.pallas import tpu as pltpu
from jax.experimental.pallas import tpu_sc as plsc

N = 10000
NPAD = 10240
E = 320000
NC = 2
NS = 16
NW = NC * NS
K = 128
CH = 80
EPW = CH * K
PE = NW * EPW
SLICE = NPAD // NS
D_HID = 64
N_CLASSES = 16


KD = 128
CHD = EPW // KD


def _deg_body(dstb, wb, out, acc, dstv, wv, zv, sem):
    c = lax.axis_index("c")
    s = lax.axis_index("s")
    wid = c * NS + s

    def z(j, _):
        zv[pl.ds(j * 16, 16)] = jnp.zeros((16,), jnp.float32)
        return 0

    lax.fori_loop(0, SLICE // 16, z, 0)
    pltpu.sync_copy(zv, acc.at[pl.ds(s * SLICE, SLICE)])
    plsc.subcore_barrier()

    pltpu.sync_copy(dstb.at[wid], dstv)
    pltpu.sync_copy(wb.at[wid], wv)

    def fire(i, _):
        pltpu.async_copy(wv.at[i], acc.at[dstv.at[i]], sem, add=True)
        return 0

    lax.fori_loop(0, CHD, fire, 0)

    def drain(i, _):
        pltpu.make_async_copy(wv.at[i], acc.at[dstv.at[i]], sem).wait()
        return 0

    lax.fori_loop(0, CHD, drain, 0)
    plsc.subcore_barrier()
    pltpu.sync_copy(acc.at[pl.ds(s * SLICE, SLICE)],
                    out.at[c, pl.ds(s * SLICE, SLICE)])


_deg = pl.kernel(
    _deg_body,
    out_type=jax.ShapeDtypeStruct((NC, NPAD), jnp.float32),
    mesh=plsc.VectorSubcoreMesh(core_axis_name="c", subcore_axis_name="s"),
    compiler_params=pltpu.CompilerParams(use_tc_tiling_on_sc=False),
    scratch_types=[
        pltpu.VMEM_SHARED((NPAD,), jnp.float32),
        pltpu.VMEM((CHD, KD), jnp.int32),
        pltpu.VMEM((CHD, KD), jnp.float32),
        pltpu.VMEM((SLICE,), jnp.float32),
        pltpu.SemaphoreType.DMA,
    ],
)


def _agg_body(D, g_hbm, srcb, dstb, wb, out, acc, srcv, dstv, wv,
              rows0, rows1, rows2, rows3, gsem, ssem):
    c = lax.axis_index("c")
    s = lax.axis_index("s")
    wid = c * NS + s

    def zr(j, _):
        for t in range(D // 16):
            rows0[j, pl.ds(t * 16, 16)] = jnp.zeros((16,), jnp.float32)
        return 0

    lax.fori_loop(0, K, zr, 0)

    def zc(r, _):
        pltpu.sync_copy(rows0, acc.at[pl.ds(s * SLICE + r * K, K), :])
        return 0

    lax.fori_loop(0, SLICE // K, zc, 0)
    plsc.subcore_barrier()

    pltpu.sync_copy(srcb.at[wid], srcv)
    pltpu.sync_copy(dstb.at[wid], dstv)
    pltpu.sync_copy(wb.at[wid], wv)

    def start_gather(i, buf):
        pltpu.async_copy(g_hbm.at[srcv.at[i]], buf, gsem)

    def wait_gather(i, buf):
        pltpu.make_async_copy(g_hbm.at[srcv.at[i]], buf, gsem).wait()

    def start_scatter(i, buf):
        pltpu.async_copy(buf, acc.at[dstv.at[i]], ssem, add=True)

    def wait_scatter(i, buf):
        pltpu.make_async_copy(buf, acc.at[dstv.at[i]], ssem).wait()

    def scale(i, buf):
        def grp(gidx, _):
            wvec = wv[i, pl.ds(gidx * 16, 16)]
            for l in range(16):
                nv = wvec[l]
                j = gidx * 16 + l
                for t in range(D // 16):
                    sl = pl.ds(t * 16, 16)
                    buf[j, sl] = buf[j, sl] * nv
            return 0

        lax.fori_loop(0, K // 16, grp, 0)

    bufs = (rows0, rows1, rows2, rows3)
    start_gather(0, bufs[0])
    start_gather(1, bufs[1])

    def quad(q, _):
        for b in range(4):
            i = q * 4 + b
            wait_gather(i, bufs[b])
            scale(i, bufs[b])
            start_scatter(i, bufs[b])
            bp = (b + 2) % 4

            @pl.when(i >= 2)
            def _():
                wait_scatter(jnp.maximum(i - 2, 0), bufs[bp])

            @pl.when(i + 2 < CH)
            def _():
                start_gather(jnp.minimum(i + 2, CH - 1), bufs[bp])
        return 0

    lax.fori_loop(0, CH // 4, quad, 0)
    wait_scatter(CH - 2, bufs[(CH - 2) % 4])
    wait_scatter(CH - 1, bufs[(CH - 1) % 4])
    plsc.subcore_barrier()
    pltpu.sync_copy(acc.at[pl.ds(s * SLICE, SLICE), :],
                    out.at[c, pl.ds(s * SLICE, SLICE), :])


def _make_agg(D):
    return pl.kernel(
        functools.partial(_agg_body, D),
        out_type=jax.ShapeDtypeStruct((NC, NPAD, D), jnp.float32),
        mesh=plsc.VectorSubcoreMesh(core_axis_name="c", subcore_axis_name="s"),
        compiler_params=pltpu.CompilerParams(use_tc_tiling_on_sc=False),
        scratch_types=[
            pltpu.VMEM_SHARED((NPAD, D), jnp.float32),
            pltpu.VMEM((CH, K), jnp.int32),
            pltpu.VMEM((CH, K), jnp.int32),
            pltpu.VMEM((CH, K), jnp.float32),
            pltpu.VMEM((K, D), jnp.float32),
            pltpu.VMEM((K, D), jnp.float32),
            pltpu.VMEM((K, D), jnp.float32),
            pltpu.VMEM((K, D), jnp.float32),
            pltpu.SemaphoreType.DMA,
            pltpu.SemaphoreType.DMA,
        ],
    )


_agg64 = _make_agg(D_HID)
_agg16 = _make_agg(N_CLASSES)


def _mms_kernel(x_ref, w_ref, dp_ref, o_ref, dc_ref):
    dc = lax.rsqrt(dp_ref[:, 0:1] + dp_ref[:, 1:2] + 1.0)
    dc_ref[...] = dc
    o_ref[...] = (jnp.dot(x_ref[...], w_ref[...],
                          preferred_element_type=jnp.float32) * dc)


def _mm_scale(x, w, dpT, bm=1000):
    m, k = x.shape
    n = w.shape[1]
    return pl.pallas_call(
        _mms_kernel,
        grid=(m // bm,),
        in_specs=[
            pl.BlockSpec((bm, k), lambda i: (i, 0)),
            pl.BlockSpec((k, n), lambda i: (0, 0)),
            pl.BlockSpec((bm, 2), lambda i: (i, 0)),
        ],
        out_specs=[
            pl.BlockSpec((bm, n), lambda i: (i, 0)),
            pl.BlockSpec((bm, 1), lambda i: (i, 0)),
        ],
        out_shape=[
            jax.ShapeDtypeStruct((m, n), jnp.float32),
            jax.ShapeDtypeStruct((m, 1), jnp.float32),
        ],
    )(x, w, dpT)


def _comb2_kernel(p0, p1, g, d, b, w2, o):
    h = jnp.maximum((p0[...] + p1[...] + g[...]) * d[...] + b[...], 0.0)
    o[...] = jnp.dot(h, w2[...], preferred_element_type=jnp.float32) * d[...]


def _comb2(p0, p1, g, dcol, b, w2, bm=1000):
    m, k = g.shape
    n = w2.shape[1]
    return pl.pallas_call(
        _comb2_kernel,
        grid=(m // bm,),
        in_specs=[
            pl.BlockSpec((bm, k), lambda i: (i, 0)),
            pl.BlockSpec((bm, k), lambda i: (i, 0)),
            pl.BlockSpec((bm, k), lambda i: (i, 0)),
            pl.BlockSpec((bm, 1), lambda i: (i, 0)),
            pl.BlockSpec((1, k), lambda i: (0, 0)),
            pl.BlockSpec((k, n), lambda i: (0, 0)),
        ],
        out_specs=pl.BlockSpec((bm, n), lambda i: (i, 0)),
        out_shape=jax.ShapeDtypeStruct((m, n), jnp.float32),
    )(p0, p1, g, dcol, b, w2)


def _final_kernel(q0, q1, g, d, b, o):
    t = (q0[...] + q1[...] + g[...]) * d[...] + b[...]
    m = jnp.max(t, axis=1, keepdims=True)
    e = t - m
    lse = jnp.log(jnp.sum(jnp.exp(e), axis=1, keepdims=True))
    o[...] = e - lse


def _final(q0, q1, g, dcol, b, bm=1000):
    m, n = g.shape
    return pl.pallas_call(
        _final_kernel,
        grid=(m // bm,),
        in_specs=[
            pl.BlockSpec((bm, n), lambda i: (i, 0)),
            pl.BlockSpec((bm, n), lambda i: (i, 0)),
            pl.BlockSpec((bm, n), lambda i: (i, 0)),
            pl.BlockSpec((bm, 1), lambda i: (i, 0)),
            pl.BlockSpec((1, n), lambda i: (0, 0)),
        ],
        out_specs=pl.BlockSpec((bm, n), lambda i: (i, 0)),
        out_shape=jax.ShapeDtypeStruct((m, n), jnp.float32),
    )(q0, q1, g, dcol, b)


def kernel(x, edge_index, edge_attr, W1, b1, W2, b2):
    pad = PE - E
    src = jnp.concatenate(
        [edge_index[0].astype(jnp.int32), jnp.zeros((pad,), jnp.int32)]
    ).reshape(NW, CH, K)
    dst = jnp.concatenate(
        [edge_index[1].astype(jnp.int32), jnp.zeros((pad,), jnp.int32)]
    ).reshape(NW, CH, K)
    wb = jnp.concatenate(
        [edge_attr, jnp.zeros((pad,), jnp.float32)]
    ).reshape(NW, CH, K)

    degp = _deg(dst.reshape(NW, CHD, KD),
                wb.reshape(NW, CHD, KD))
    g1, dcol = _mm_scale(x, W1, degp.T[:N])
    p = _agg64(g1, src, dst, wb)
    g2 = _comb2(p[0, :N], p[1, :N], g1, dcol, b1.reshape(1, -1), W2)
    q = _agg16(g2, src, dst, wb)
    return _final(q[0, :N], q[1, :N], g2, dcol, b2.reshape(1, -1))
